# Initial kernel scaffold; baseline (speedup 1.0000x reference)
import functools

import jax
import jax.numpy as jnp
from jax import lax
from jax.experimental import pallas as pl
from jax.experimental.pallas import tpu as pltpu
from jax.experimental.pallas import tpu_sc as plsc

F32 = jnp.float32
I32 = jnp.int32

N = 50000
E = 800000
NP = 50176
NPC = NP // 2
ACC_ROWS = NPC + 16
DUMP = NPC
NSC = 16
EPS = 1e-5
W = 128
RB = 512
NBLK = NP // RB


def _mesh():
    return plsc.VectorSubcoreMesh(core_axis_name="c", subcore_axis_name="s")


def _zero_rows(ref, nrows):
    @pl.loop(0, nrows)
    def _(i):
        @pl.loop(0, 4)
        def _(j):
            ref[i, pl.ds(j * 16, 16)] = jnp.zeros((16,), F32)


def _sc_degree(dst):
    EC = E // NSC
    NWIN = EC // W
    TAIL = EC - NWIN * W
    SLC = NPC // NSC

    @functools.partial(
        pl.kernel,
        out_type=jax.ShapeDtypeStruct((NP,), F32),
        mesh=_mesh(),
        scratch_types=[
            pltpu.VMEM((W,), I32),
            pltpu.VMEM((TAIL,), I32),
            pltpu.VMEM((W,), F32),
            pltpu.VMEM((SLC,), F32),
            pltpu.VMEM((SLC,), F32),
            pltpu.VMEM_SHARED((NP,), F32),
        ],
    )
    def k(dst_hbm, dinv_hbm, idxb, idxt, oneb, cntb, outb, acc):
        c = lax.axis_index("c")
        s = lax.axis_index("s")
        @pl.loop(0, W // 16)
        def _(i):
            oneb[pl.ds(i * 16, 16)] = jnp.ones((16,), F32)

        @pl.loop(0, SLC // 16)
        def _(i):
            cntb[pl.ds(i * 16, 16)] = jnp.zeros((16,), F32)

        pltpu.sync_copy(cntb, acc.at[pl.ds(s * (NP // NSC), SLC)])
        pltpu.sync_copy(cntb, acc.at[pl.ds(s * (NP // NSC) + SLC, NP // NSC - SLC)])
        plsc.subcore_barrier()

        base = s * EC

        @pl.loop(0, NWIN)
        def _(w):
            pltpu.sync_copy(dst_hbm.at[pl.ds(base + w * W, W)], idxb)
            pltpu.sync_copy(oneb, acc.at[idxb], add=True)

        pltpu.sync_copy(dst_hbm.at[pl.ds(base + NWIN * W, TAIL)], idxt)
        pltpu.sync_copy(oneb.at[pl.ds(0, TAIL)], acc.at[idxt], add=True)
        plsc.subcore_barrier()

        r0 = c * NPC + s * SLC
        pltpu.sync_copy(acc.at[pl.ds(r0, SLC)], cntb)

        @pl.loop(0, SLC // 16)
        def _(i):
            d = cntb[pl.ds(i * 16, 16)] + 1.0
            iv = plsc.bitcast(d, I32)
            iv = 0x5F3759DF - lax.shift_right_arithmetic(iv, 1)
            y = plsc.bitcast(iv, F32)
            y = y * (1.5 - 0.5 * d * y * y)
            y = y * (1.5 - 0.5 * d * y * y)
            y = y * (1.5 - 0.5 * d * y * y)
            outb[pl.ds(i * 16, 16)] = y

        pltpu.sync_copy(outb, dinv_hbm.at[pl.ds(r0, SLC)])

    return k(dst)


def _sc_conv(src, dst, y_groups):
    FG = len(y_groups)
    EC = E // NSC
    NWIN = EC // W
    TAIL = EC - NWIN * W
    ZR = ACC_ROWS // NSC + 1
    OR = NPC // NSC

    @functools.partial(
        pl.kernel,
        out_type=[jax.ShapeDtypeStruct((NP, 64), F32) for _ in range(FG)],
        mesh=_mesh(),
        scratch_types=[
            pltpu.VMEM((W,), I32),
            pltpu.VMEM((TAIL,), I32),
            pltpu.VMEM((W,), I32),
            pltpu.VMEM((TAIL,), I32),
            pltpu.VMEM((W,), I32),
            pltpu.VMEM((TAIL,), I32),
            pltpu.VMEM((W, 64), F32),
            pltpu.VMEM((TAIL, 64), F32),
            pltpu.VMEM((W, 64), F32),
            pltpu.VMEM_SHARED((ACC_ROWS, 64), F32),
        ],
    )
    def k(src_hbm, dst_hbm, *rest):
        y_hbms = rest[:FG]
        out_hbms = rest[FG:2 * FG]
        (srcb, srct, dstb, dstt, dlb, dlt, rowb, rowt, zrow, acc) = rest[2 * FG:]
        c = lax.axis_index("c")
        s = lax.axis_index("s")
        lo = c * NPC
        base = s * EC

        _zero_rows(zrow, W)

        pltpu.sync_copy(dst_hbm.at[pl.ds(base + NWIN * W, TAIL)], dstt)

        @pl.loop(0, TAIL // 16)
        def _(i):
            dv = dstt[pl.ds(i * 16, 16)]
            m = (dv >= lo) & (dv < lo + NPC)
            dlt[pl.ds(i * 16, 16)] = jnp.where(m, dv - lo, DUMP)

        for g in range(FG):
            z0 = s * ZR
            nfull = ZR // W

            @pl.loop(0, nfull)
            def _(j, z0=z0):
                pltpu.sync_copy(zrow, acc.at[pl.ds(z0 + j * W, W)])

            rem = ZR - nfull * W
            pltpu.sync_copy(zrow.at[pl.ds(0, rem)],
                            acc.at[pl.ds(z0 + nfull * W, rem)])
            plsc.subcore_barrier()

            @pl.loop(0, NWIN)
            def _(w, g=g):
                pltpu.sync_copy(src_hbm.at[pl.ds(base + w * W, W)], srcb)
                pltpu.sync_copy(dst_hbm.at[pl.ds(base + w * W, W)], dstb)

                @pl.loop(0, W // 16)
                def _(i):
                    dv = dstb[pl.ds(i * 16, 16)]
                    m = (dv >= lo) & (dv < lo + NPC)
                    dlb[pl.ds(i * 16, 16)] = jnp.where(m, dv - lo, DUMP)

                pltpu.sync_copy(y_hbms[g].at[srcb], rowb)
                pltpu.sync_copy(rowb, acc.at[dlb], add=True)

            pltpu.sync_copy(src_hbm.at[pl.ds(base + NWIN * W, TAIL)], srct)
            pltpu.sync_copy(y_hbms[g].at[srct], rowt)
            pltpu.sync_copy(rowt, acc.at[dlt], add=True)
            plsc.subcore_barrier()

            pltpu.sync_copy(acc.at[pl.ds(s * OR, OR)],
                            out_hbms[g].at[pl.ds(lo + s * OR, OR)])
            if g != FG - 1:
                plsc.subcore_barrier()

    return k(src, dst, *y_groups)


def _row_spec(width):
    return pl.BlockSpec((RB, width), lambda i: (i, 0))


def _full_spec(shape):
    return pl.BlockSpec(shape, lambda i: tuple(0 for _ in shape))


def _tc_first(xp, w1p, dinv):
    def body(x_ref, w_ref, d_ref, y0_ref, y1_ref):
        xb = jnp.clip(x_ref[...], -0.4, 0.4)
        y = d_ref[...] * jnp.dot(xb, w_ref[...], preferred_element_type=F32)
        y0_ref[...] = y[:, :64]
        y1_ref[...] = y[:, 64:]

    return pl.pallas_call(
        body,
        grid=(NBLK,),
        in_specs=[_row_spec(128), _full_spec((128, 128)), _row_spec(1)],
        out_specs=[_row_spec(64), _row_spec(64)],
        out_shape=[jax.ShapeDtypeStruct((NP, 64), F32)] * 2,
    )(xp, w1p, dinv)


def _tc_mid(s_groups, y_groups, dinv, bp, wnextp, fg_out):
    fg_in = len(s_groups)
    dout = 64 * fg_out

    def body(*refs):
        s_refs = refs[:fg_in]
        y_refs = refs[fg_in:2 * fg_in]
        d_ref, b_ref, w_ref = refs[2 * fg_in:2 * fg_in + 3]
        out_refs = refs[2 * fg_in + 3:]
        sb = jnp.concatenate([r[...] for r in s_refs], axis=1)
        yb = jnp.concatenate([r[...] for r in y_refs], axis=1)
        h = jax.nn.relu(d_ref[...] * (sb + yb) + b_ref[...])
        z = d_ref[...] * jnp.dot(h, w_ref[...], preferred_element_type=F32)
        for g in range(fg_out):
            out_refs[g][...] = z[:, g * 64:(g + 1) * 64]

    return pl.pallas_call(
        body,
        grid=(NBLK,),
        in_specs=([_row_spec(64)] * (2 * fg_in)
                  + [_row_spec(1), _full_spec((1, 64 * fg_in)),
                     _full_spec((64 * fg_in, dout))]),
        out_specs=[_row_spec(64)] * fg_out,
        out_shape=[jax.ShapeDtypeStruct((NP, 64), F32)] * fg_out,
    )(*s_groups, *y_groups, dinv, bp, wnextp)


def _tc_head1(s_groups, y_groups, dinv, b3p, wf1, bf1, wf2, bf2):
    fg = len(s_groups)

    def body(*refs):
        s_refs = refs[:fg]
        y_refs = refs[fg:2 * fg]
        d_ref, b3_ref, w1_ref, b1_ref, w2_ref, b2_ref = refs[2 * fg:2 * fg + 6]
        f2_ref, st_ref, accum = refs[2 * fg + 6:]
        i = pl.program_id(0)

        @pl.when(i == 0)
        def _():
            accum[...] = jnp.zeros((8, 128), F32)

        sb = jnp.concatenate([r[...] for r in s_refs], axis=1)
        yb = jnp.concatenate([r[...] for r in y_refs], axis=1)
        h3 = jax.nn.relu(d_ref[...] * (sb + yb) + b3_ref[...])
        f1 = jax.nn.relu(
            jnp.dot(h3, w1_ref[...], preferred_element_type=F32) + b1_ref[...])
        f2 = jnp.dot(f1, w2_ref[...], preferred_element_type=F32) + b2_ref[...]
        f2_ref[...] = f2
        rows = i * RB + lax.broadcasted_iota(I32, (RB, 128), 0)
        f2m = jnp.where(rows < N, f2, 0.0)
        ssum = jnp.sum(f2m, axis=0, keepdims=True)
        ssq = jnp.sum(f2m * f2m, axis=0, keepdims=True)
        accum[0:1, :] = accum[0:1, :] + ssum
        accum[1:2, :] = accum[1:2, :] + ssq

        @pl.when(i == NBLK - 1)
        def _():
            st_ref[...] = accum[...]

    return pl.pallas_call(
        body,
        grid=(NBLK,),
        in_specs=([_row_spec(64)] * (2 * fg)
                  + [_row_spec(1), _full_spec((1, 256)),
                     _full_spec((256, 256)), _full_spec((1, 256)),
                     _full_spec((256, 128)), _full_spec((1, 128))]),
        out_specs=[_row_spec(128), _full_spec((8, 128))],
        out_shape=[jax.ShapeDtypeStruct((NP, 128), F32),
                   jax.ShapeDtypeStruct((8, 128), F32)],
        scratch_shapes=[pltpu.VMEM((8, 128), F32)],
    )(*s_groups, *y_groups, dinv, b3p, wf1, bf1, wf2, bf2)


def _tc_head2(f2, stats, gammap, betap, wf3p, bf3p):
    def body(f_ref, st_ref, g_ref, be_ref, w_ref, b_ref, o_ref):
        mean = st_ref[0:1, :] * (1.0 / N)
        var = st_ref[1:2, :] * (1.0 / N) - mean * mean
        xn = (f_ref[...] - mean) * lax.rsqrt(var + EPS) * g_ref[...] + be_ref[...]
        r = jax.nn.relu(xn)
        z = jnp.dot(r, w_ref[...], preferred_element_type=F32) + b_ref[...]
        col = lax.broadcasted_iota(I32, (RB, 128), 1)
        zm = jnp.where(col < 19, z, -1e30)
        mx = jnp.max(zm, axis=1, keepdims=True)
        lse = jnp.log(jnp.sum(jnp.exp(zm - mx), axis=1, keepdims=True))
        o_ref[...] = zm - mx - lse

    return pl.pallas_call(
        body,
        grid=(NBLK,),
        in_specs=[_row_spec(128), _full_spec((8, 128)), _full_spec((1, 128)),
                  _full_spec((1, 128)), _full_spec((128, 128)),
                  _full_spec((1, 128))],
        out_specs=_row_spec(128),
        out_shape=jax.ShapeDtypeStruct((NP, 128), F32),
    )(f2, stats, gammap, betap, wf3p, bf3p)


def _pad2(a, r, c):
    return jnp.pad(a, ((0, r - a.shape[0]), (0, c - a.shape[1])))


def kernel(x, edge_index, W1, b1, W2, b2, W3, b3, Wf1, bf1, Wf2, bf2,
           gamma, beta, Wf3, bf3):
    src = edge_index[0]
    dst = edge_index[1]

    xp = _pad2(x, NP, 128)
    w1p = _pad2(W1, 128, 128)
    w2p = _pad2(W2, 128, 128)
    w3p = _pad2(W3, 128, 256)
    b1p = jnp.pad(b1, (0, 28)).reshape(1, 128)
    b2p = jnp.pad(b2, (0, 28)).reshape(1, 128)
    b3p = b3.reshape(1, 256)
    bf1p = bf1.reshape(1, 256)
    bf2p = bf2.reshape(1, 128)
    gammap = gamma.reshape(1, 128)
    betap = beta.reshape(1, 128)
    wf3p = _pad2(Wf3, 128, 128)
    bf3p = jnp.pad(bf3, (0, 109)).reshape(1, 128)

    dinv = _sc_degree(dst).reshape(NP, 1)

    y1 = _tc_first(xp, w1p, dinv)
    s1 = _sc_conv(src, dst, y1)
    y2 = _tc_mid(s1, y1, dinv, b1p, w2p, 2)
    s2 = _sc_conv(src, dst, y2)
    y3 = _tc_mid(s2, y2, dinv, b2p, w3p, 4)
    s3 = _sc_conv(src, dst, y3)
    f2, stats = _tc_head1(s3, y3, dinv, b3p, Wf1, bf1p, Wf2, bf2p)
    out = _tc_head2(f2, stats, gammap, betap, wf3p, bf3p)
    return out[:N, :19]


# speedup vs baseline: 4.3026x; 4.3026x over previous
import dataclasses
import functools

import jax
import jax.numpy as jnp
from jax import lax
from jax.experimental import pallas as pl
from jax.experimental.pallas import tpu as pltpu
from jax.experimental.pallas import tpu_sc as plsc

F32 = jnp.float32
I32 = jnp.int32

N = 50000
E = 800000
NP = 50176
NPC = NP // 2
ACC_ROWS = NPC + 16
DUMP = NPC
NSC = 16
EPS = 1e-5
W = 128
RB = 512
NBLK = NP // RB


def _mesh():
    return plsc.VectorSubcoreMesh(core_axis_name="c", subcore_axis_name="s")


def _sc_params():
    return pltpu.CompilerParams(needs_layout_passes=False,
                                use_tc_tiling_on_sc=False)


def _zero_rows(ref, nrows):
    @pl.loop(0, nrows)
    def _(i):
        @pl.loop(0, 4)
        def _(j):
            ref[i, pl.ds(j * 16, 16)] = jnp.zeros((16,), F32)


def _sc_degree(dst):
    EC = E // NSC
    NWIN = EC // W
    TAIL = EC - NWIN * W
    SLC = NPC // NSC

    @functools.partial(
        pl.kernel,
        out_type=jax.ShapeDtypeStruct((NP,), F32),
        mesh=_mesh(),
        compiler_params=_sc_params(),
        scratch_types=[
            pltpu.VMEM((W,), I32),
            pltpu.VMEM((TAIL,), I32),
            pltpu.VMEM((W,), F32),
            pltpu.VMEM((SLC,), F32),
            pltpu.VMEM((SLC,), F32),
            pltpu.VMEM_SHARED((NP,), F32),
        ],
    )
    def k(dst_hbm, dinv_hbm, idxb, idxt, oneb, cntb, outb, acc):
        c = lax.axis_index("c")
        s = lax.axis_index("s")
        @pl.loop(0, W // 16)
        def _(i):
            oneb[pl.ds(i * 16, 16)] = jnp.ones((16,), F32)

        @pl.loop(0, SLC // 16)
        def _(i):
            cntb[pl.ds(i * 16, 16)] = jnp.zeros((16,), F32)

        pltpu.sync_copy(cntb, acc.at[pl.ds(s * (NP // NSC), SLC)])
        pltpu.sync_copy(cntb, acc.at[pl.ds(s * (NP // NSC) + SLC, NP // NSC - SLC)])
        plsc.subcore_barrier()

        base = s * EC

        @pl.loop(0, NWIN)
        def _(w):
            pltpu.sync_copy(dst_hbm.at[pl.ds(base + w * W, W)], idxb)
            pltpu.sync_copy(oneb, acc.at[idxb], add=True)

        pltpu.sync_copy(dst_hbm.at[pl.ds(base + NWIN * W, TAIL)], idxt)
        pltpu.sync_copy(oneb.at[pl.ds(0, TAIL)], acc.at[idxt], add=True)
        plsc.subcore_barrier()

        r0 = c * NPC + s * SLC
        pltpu.sync_copy(acc.at[pl.ds(r0, SLC)], cntb)

        @pl.loop(0, SLC // 16)
        def _(i):
            d = cntb[pl.ds(i * 16, 16)] + 1.0
            iv = plsc.bitcast(d, I32)
            iv = 0x5F3759DF - lax.shift_right_arithmetic(iv, 1)
            y = plsc.bitcast(iv, F32)
            y = y * (1.5 - 0.5 * d * y * y)
            y = y * (1.5 - 0.5 * d * y * y)
            y = y * (1.5 - 0.5 * d * y * y)
            outb[pl.ds(i * 16, 16)] = y

        pltpu.sync_copy(outb, dinv_hbm.at[pl.ds(r0, SLC)])

    return k(dst)


def _sc_conv(src, dst, y_groups):
    FG = len(y_groups)
    EC = E // NSC
    NWIN = EC // W
    TAIL = EC - NWIN * W
    ZR = ACC_ROWS // NSC + 1
    OR = NPC // NSC

    @functools.partial(
        pl.kernel,
        out_type=[jax.ShapeDtypeStruct((NP, 64), F32) for _ in range(FG)],
        mesh=_mesh(),
        compiler_params=_sc_params(),
        scratch_types=[
            pltpu.VMEM((W,), I32),
            pltpu.VMEM((TAIL,), I32),
            pltpu.VMEM((W,), I32),
            pltpu.VMEM((TAIL,), I32),
            pltpu.VMEM((W,), I32),
            pltpu.VMEM((TAIL,), I32),
            pltpu.VMEM((W, 64), F32),
            pltpu.VMEM((TAIL, 64), F32),
            pltpu.VMEM((W, 64), F32),
            pltpu.VMEM_SHARED((ACC_ROWS, 64), F32),
        ],
    )
    def k(src_hbm, dst_hbm, *rest):
        y_hbms = rest[:FG]
        out_hbms = rest[FG:2 * FG]
        (srcb, srct, dstb, dstt, dlb, dlt, rowb, rowt, zrow, acc) = rest[2 * FG:]
        c = lax.axis_index("c")
        s = lax.axis_index("s")
        lo = c * NPC
        base = s * EC

        _zero_rows(zrow, W)

        pltpu.sync_copy(dst_hbm.at[pl.ds(base + NWIN * W, TAIL)], dstt)

        @pl.loop(0, TAIL // 16)
        def _(i):
            dv = dstt[pl.ds(i * 16, 16)]
            m = (dv >= lo) & (dv < lo + NPC)
            dlt[pl.ds(i * 16, 16)] = jnp.where(m, dv - lo, DUMP)

        for g in range(FG):
            z0 = s * ZR
            nfull = ZR // W

            @pl.loop(0, nfull)
            def _(j, z0=z0):
                pltpu.sync_copy(zrow, acc.at[pl.ds(z0 + j * W, W)])

            rem = ZR - nfull * W
            pltpu.sync_copy(zrow.at[pl.ds(0, rem)],
                            acc.at[pl.ds(z0 + nfull * W, rem)])
            plsc.subcore_barrier()

            @pl.loop(0, NWIN)
            def _(w, g=g):
                pltpu.sync_copy(src_hbm.at[pl.ds(base + w * W, W)], srcb)
                pltpu.sync_copy(dst_hbm.at[pl.ds(base + w * W, W)], dstb)

                @pl.loop(0, W // 16)
                def _(i):
                    dv = dstb[pl.ds(i * 16, 16)]
                    m = (dv >= lo) & (dv < lo + NPC)
                    dlb[pl.ds(i * 16, 16)] = jnp.where(m, dv - lo, DUMP)

                pltpu.sync_copy(y_hbms[g].at[srcb], rowb)
                pltpu.sync_copy(rowb, acc.at[dlb], add=True)

            pltpu.sync_copy(src_hbm.at[pl.ds(base + NWIN * W, TAIL)], srct)
            pltpu.sync_copy(y_hbms[g].at[srct], rowt)
            pltpu.sync_copy(rowt, acc.at[dlt], add=True)
            plsc.subcore_barrier()

            pltpu.sync_copy(acc.at[pl.ds(s * OR, OR)],
                            out_hbms[g].at[pl.ds(lo + s * OR, OR)])
            if g != FG - 1:
                plsc.subcore_barrier()

    return k(src, dst, *y_groups)


def _row_spec(width):
    return pl.BlockSpec((RB, width), lambda i: (i, 0))


def _full_spec(shape):
    return pl.BlockSpec(shape, lambda i: tuple(0 for _ in shape))


def _tc_first(xp, w1p, dinv):
    def body(x_ref, w_ref, d_ref, y0_ref, y1_ref):
        xb = jnp.clip(x_ref[...], -0.4, 0.4)
        y = d_ref[...] * jnp.dot(xb, w_ref[...], preferred_element_type=F32)
        y0_ref[...] = y[:, :64]
        y1_ref[...] = y[:, 64:]

    return pl.pallas_call(
        body,
        grid=(NBLK,),
        in_specs=[_row_spec(128), _full_spec((128, 128)), _row_spec(1)],
        out_specs=[_row_spec(64), _row_spec(64)],
        out_shape=[jax.ShapeDtypeStruct((NP, 64), F32)] * 2,
    )(xp, w1p, dinv)


def _tc_mid(s_groups, y_groups, dinv, bp, wnextp, fg_out):
    fg_in = len(s_groups)
    dout = 64 * fg_out

    def body(*refs):
        s_refs = refs[:fg_in]
        y_refs = refs[fg_in:2 * fg_in]
        d_ref, b_ref, w_ref = refs[2 * fg_in:2 * fg_in + 3]
        out_refs = refs[2 * fg_in + 3:]
        sb = jnp.concatenate([r[...] for r in s_refs], axis=1)
        yb = jnp.concatenate([r[...] for r in y_refs], axis=1)
        h = jax.nn.relu(d_ref[...] * (sb + yb) + b_ref[...])
        z = d_ref[...] * jnp.dot(h, w_ref[...], preferred_element_type=F32)
        for g in range(fg_out):
            out_refs[g][...] = z[:, g * 64:(g + 1) * 64]

    return pl.pallas_call(
        body,
        grid=(NBLK,),
        in_specs=([_row_spec(64)] * (2 * fg_in)
                  + [_row_spec(1), _full_spec((1, 64 * fg_in)),
                     _full_spec((64 * fg_in, dout))]),
        out_specs=[_row_spec(64)] * fg_out,
        out_shape=[jax.ShapeDtypeStruct((NP, 64), F32)] * fg_out,
    )(*s_groups, *y_groups, dinv, bp, wnextp)


def _tc_head1(s_groups, y_groups, dinv, b3p, wf1, bf1, wf2, bf2):
    fg = len(s_groups)

    def body(*refs):
        s_refs = refs[:fg]
        y_refs = refs[fg:2 * fg]
        d_ref, b3_ref, w1_ref, b1_ref, w2_ref, b2_ref = refs[2 * fg:2 * fg + 6]
        f2_ref, st_ref, accum = refs[2 * fg + 6:]
        i = pl.program_id(0)

        @pl.when(i == 0)
        def _():
            accum[...] = jnp.zeros((8, 128), F32)

        sb = jnp.concatenate([r[...] for r in s_refs], axis=1)
        yb = jnp.concatenate([r[...] for r in y_refs], axis=1)
        h3 = jax.nn.relu(d_ref[...] * (sb + yb) + b3_ref[...])
        f1 = jax.nn.relu(
            jnp.dot(h3, w1_ref[...], preferred_element_type=F32) + b1_ref[...])
        f2 = jnp.dot(f1, w2_ref[...], preferred_element_type=F32) + b2_ref[...]
        f2_ref[...] = f2
        rows = i * RB + lax.broadcasted_iota(I32, (RB, 128), 0)
        f2m = jnp.where(rows < N, f2, 0.0)
        ssum = jnp.sum(f2m, axis=0, keepdims=True)
        ssq = jnp.sum(f2m * f2m, axis=0, keepdims=True)
        accum[0:1, :] = accum[0:1, :] + ssum
        accum[1:2, :] = accum[1:2, :] + ssq

        @pl.when(i == NBLK - 1)
        def _():
            st_ref[...] = accum[...]

    return pl.pallas_call(
        body,
        grid=(NBLK,),
        in_specs=([_row_spec(64)] * (2 * fg)
                  + [_row_spec(1), _full_spec((1, 256)),
                     _full_spec((256, 256)), _full_spec((1, 256)),
                     _full_spec((256, 128)), _full_spec((1, 128))]),
        out_specs=[_row_spec(128), _full_spec((8, 128))],
        out_shape=[jax.ShapeDtypeStruct((NP, 128), F32),
                   jax.ShapeDtypeStruct((8, 128), F32)],
        scratch_shapes=[pltpu.VMEM((8, 128), F32)],
    )(*s_groups, *y_groups, dinv, b3p, wf1, bf1, wf2, bf2)


def _tc_head2(f2, stats, gammap, betap, wf3p, bf3p):
    def body(f_ref, st_ref, g_ref, be_ref, w_ref, b_ref, o_ref):
        mean = st_ref[0:1, :] * (1.0 / N)
        var = st_ref[1:2, :] * (1.0 / N) - mean * mean
        xn = (f_ref[...] - mean) * lax.rsqrt(var + EPS) * g_ref[...] + be_ref[...]
        r = jax.nn.relu(xn)
        z = jnp.dot(r, w_ref[...], preferred_element_type=F32) + b_ref[...]
        col = lax.broadcasted_iota(I32, (RB, 128), 1)
        zm = jnp.where(col < 19, z, -1e30)
        mx = jnp.max(zm, axis=1, keepdims=True)
        lse = jnp.log(jnp.sum(jnp.exp(zm - mx), axis=1, keepdims=True))
        o_ref[...] = zm - mx - lse

    return pl.pallas_call(
        body,
        grid=(NBLK,),
        in_specs=[_row_spec(128), _full_spec((8, 128)), _full_spec((1, 128)),
                  _full_spec((1, 128)), _full_spec((128, 128)),
                  _full_spec((1, 128))],
        out_specs=_row_spec(128),
        out_shape=jax.ShapeDtypeStruct((NP, 128), F32),
    )(f2, stats, gammap, betap, wf3p, bf3p)


def _pad2(a, r, c):
    return jnp.pad(a, ((0, r - a.shape[0]), (0, c - a.shape[1])))


def kernel(x, edge_index, W1, b1, W2, b2, W3, b3, Wf1, bf1, Wf2, bf2,
           gamma, beta, Wf3, bf3):
    src = edge_index[0]
    dst = edge_index[1]

    xp = _pad2(x, NP, 128)
    w1p = _pad2(W1, 128, 128)
    w2p = _pad2(W2, 128, 128)
    w3p = _pad2(W3, 128, 256)
    b1p = jnp.pad(b1, (0, 28)).reshape(1, 128)
    b2p = jnp.pad(b2, (0, 28)).reshape(1, 128)
    b3p = b3.reshape(1, 256)
    bf1p = bf1.reshape(1, 256)
    bf2p = bf2.reshape(1, 128)
    gammap = gamma.reshape(1, 128)
    betap = beta.reshape(1, 128)
    wf3p = _pad2(Wf3, 128, 128)
    bf3p = jnp.pad(bf3, (0, 109)).reshape(1, 128)

    dinv = _sc_degree(dst).reshape(NP, 1)

    y1 = _tc_first(xp, w1p, dinv)
    s1 = _sc_conv(src, dst, y1)
    y2 = _tc_mid(s1, y1, dinv, b1p, w2p, 2)
    s2 = _sc_conv(src, dst, y2)
    y3 = _tc_mid(s2, y2, dinv, b2p, w3p, 4)
    s3 = _sc_conv(src, dst, y3)
    f2, stats = _tc_head1(s3, y3, dinv, b3p, Wf1, bf1p, Wf2, bf2p)
    out = _tc_head2(f2, stats, gammap, betap, wf3p, bf3p)
    return out[:N, :19]


# speedup vs baseline: 6.6640x; 1.5488x over previous
import dataclasses
import functools

import jax
import jax.numpy as jnp
from jax import lax
from jax.experimental import pallas as pl
from jax.experimental.pallas import tpu as pltpu
from jax.experimental.pallas import tpu_sc as plsc

F32 = jnp.float32
I32 = jnp.int32

N = 50000
E = 800000
NP = 50176
NPC = NP // 2
ACC_ROWS = NPC + 2
DUMP = NPC
NSC = 16
EPS = 1e-5
W = 128
RB = 512
NBLK = NP // RB


def _mesh():
    return plsc.VectorSubcoreMesh(core_axis_name="c", subcore_axis_name="s")


def _sc_params():
    return pltpu.CompilerParams(needs_layout_passes=False,
                                use_tc_tiling_on_sc=False)


def _zero_rows(ref, nrows):
    @pl.loop(0, nrows)
    def _(i):
        @pl.loop(0, 4)
        def _(j):
            ref[i, pl.ds(j * 16, 16)] = jnp.zeros((16,), F32)


def _sc_degree(dst):
    EC = E // NSC
    NWIN = EC // W
    TAIL = EC - NWIN * W
    SLC = NPC // NSC

    @functools.partial(
        pl.kernel,
        out_type=jax.ShapeDtypeStruct((NP,), F32),
        mesh=_mesh(),
        compiler_params=_sc_params(),
        scratch_types=[
            pltpu.VMEM((W,), I32),
            pltpu.VMEM((TAIL,), I32),
            pltpu.VMEM((W,), F32),
            pltpu.VMEM((SLC,), F32),
            pltpu.VMEM((SLC,), F32),
            pltpu.VMEM_SHARED((NP,), F32),
        ],
    )
    def k(dst_hbm, dinv_hbm, idxb, idxt, oneb, cntb, outb, acc):
        c = lax.axis_index("c")
        s = lax.axis_index("s")
        @pl.loop(0, W // 16)
        def _(i):
            oneb[pl.ds(i * 16, 16)] = jnp.ones((16,), F32)

        @pl.loop(0, SLC // 16)
        def _(i):
            cntb[pl.ds(i * 16, 16)] = jnp.zeros((16,), F32)

        pltpu.sync_copy(cntb, acc.at[pl.ds(s * (NP // NSC), SLC)])
        pltpu.sync_copy(cntb, acc.at[pl.ds(s * (NP // NSC) + SLC, NP // NSC - SLC)])
        plsc.subcore_barrier()

        base = s * EC

        @pl.loop(0, NWIN)
        def _(w):
            pltpu.sync_copy(dst_hbm.at[pl.ds(base + w * W, W)], idxb)
            pltpu.sync_copy(oneb, acc.at[idxb], add=True)

        pltpu.sync_copy(dst_hbm.at[pl.ds(base + NWIN * W, TAIL)], idxt)
        pltpu.sync_copy(oneb.at[pl.ds(0, TAIL)], acc.at[idxt], add=True)
        plsc.subcore_barrier()

        r0 = c * NPC + s * SLC
        pltpu.sync_copy(acc.at[pl.ds(r0, SLC)], cntb)

        @pl.loop(0, SLC // 16)
        def _(i):
            d = cntb[pl.ds(i * 16, 16)] + 1.0
            iv = plsc.bitcast(d, I32)
            iv = 0x5F3759DF - lax.shift_right_arithmetic(iv, 1)
            y = plsc.bitcast(iv, F32)
            y = y * (1.5 - 0.5 * d * y * y)
            y = y * (1.5 - 0.5 * d * y * y)
            y = y * (1.5 - 0.5 * d * y * y)
            outb[pl.ds(i * 16, 16)] = y

        pltpu.sync_copy(outb, dinv_hbm.at[pl.ds(r0, SLC)])

    return k(dst)


def _sc_conv(src, dst, y_groups):
    FG = len(y_groups)
    EC = E // NSC
    NWIN = EC // W
    PAIRS = NWIN // 2
    TAIL = EC - NWIN * W
    OR = NPC // NSC

    @functools.partial(
        pl.kernel,
        out_type=[jax.ShapeDtypeStruct((NP, 64), F32) for _ in range(FG)],
        mesh=_mesh(),
        compiler_params=_sc_params(),
        scratch_types=[
            pltpu.VMEM((2, W), I32),
            pltpu.VMEM((2, W), I32),
            pltpu.VMEM((2, W), I32),
            pltpu.VMEM((2, W, 64), F32),
            pltpu.VMEM((TAIL,), I32),
            pltpu.VMEM((TAIL,), I32),
            pltpu.VMEM((TAIL,), I32),
            pltpu.VMEM((TAIL, 64), F32),
            pltpu.VMEM((W, 64), F32),
            pltpu.VMEM_SHARED((ACC_ROWS, 64), F32),
            pltpu.SemaphoreType.DMA,
            pltpu.SemaphoreType.DMA,
            pltpu.SemaphoreType.DMA,
            pltpu.SemaphoreType.DMA,
            pltpu.SemaphoreType.DMA,
            pltpu.SemaphoreType.DMA,
        ],
    )
    def k(src_hbm, dst_hbm, *rest):
        y_hbms = rest[:FG]
        out_hbms = rest[FG:2 * FG]
        (srcb, dstb, dlb, rowb, srct, dstt, dlt, rowt, zrow, acc,
         i0, i1, g0, g1, s0, s1) = rest[2 * FG:]
        isems, gsems, ssems = (i0, i1), (g0, g1), (s0, s1)
        c = lax.axis_index("c")
        s = lax.axis_index("s")
        lo = c * NPC
        base = s * EC

        _zero_rows(zrow, W)

        pltpu.sync_copy(dst_hbm.at[pl.ds(base + NWIN * W, TAIL)], dstt)

        @pl.loop(0, TAIL // 16)
        def _(i):
            dv = dstt[pl.ds(i * 16, 16)]
            m = (dv >= lo) & (dv < lo + NPC)
            dlt[pl.ds(i * 16, 16)] = jnp.where(m, dv - lo, DUMP)

        def idx_start(w, b):
            pltpu.async_copy(src_hbm.at[pl.ds(base + w * W, W)],
                             srcb.at[b], isems[b])
            pltpu.async_copy(dst_hbm.at[pl.ds(base + w * W, W)],
                             dstb.at[b], isems[b])

        def idx_wait(b):
            pltpu.make_async_copy(src_hbm.at[pl.ds(base, W)],
                                  srcb.at[b], isems[b]).wait()
            pltpu.make_async_copy(dst_hbm.at[pl.ds(base, W)],
                                  dstb.at[b], isems[b]).wait()

        def scat_wait(b):
            pltpu.make_async_copy(rowb.at[b], acc.at[dlb.at[b]],
                                  ssems[b]).wait()

        for g in range(FG):
            z0 = s * OR
            nfull = OR // W

            @pl.loop(0, nfull)
            def _(j, z0=z0):
                pltpu.sync_copy(zrow, acc.at[pl.ds(z0 + j * W, W)])

            rem = OR - nfull * W
            pltpu.sync_copy(zrow.at[pl.ds(0, rem)],
                            acc.at[pl.ds(z0 + nfull * W, rem)])
            plsc.subcore_barrier()

            for b in (0, 1):
                @pl.loop(0, W // 16)
                def _(i, b=b):
                    dlb[b, pl.ds(i * 16, 16)] = jnp.full((16,), DUMP, I32)

                pltpu.async_copy(zrow, acc.at[dlb.at[b]], ssems[b], add=True)
                idx_start(b, b)

            @pl.loop(0, PAIRS)
            def _(p, g=g):
                for b in (0, 1):
                    w = 2 * p + b
                    idx_wait(b)
                    scat_wait(b)

                    @pl.loop(0, W // 16)
                    def _(i, b=b):
                        dv = dstb[b, pl.ds(i * 16, 16)]
                        m = (dv >= lo) & (dv < lo + NPC)
                        dlb[b, pl.ds(i * 16, 16)] = jnp.where(m, dv - lo, DUMP)

                    pltpu.async_copy(y_hbms[g].at[srcb.at[b]],
                                     rowb.at[b], gsems[b])
                    pltpu.make_async_copy(y_hbms[g].at[srcb.at[b]],
                                          rowb.at[b], gsems[b]).wait()

                    @pl.when(w + 2 < NWIN)
                    def _(w=w, b=b):
                        idx_start(w + 2, b)

                    pltpu.async_copy(rowb.at[b], acc.at[dlb.at[b]],
                                     ssems[b], add=True)

            scat_wait(0)
            scat_wait(1)

            pltpu.sync_copy(src_hbm.at[pl.ds(base + NWIN * W, TAIL)], srct)
            pltpu.sync_copy(y_hbms[g].at[srct], rowt)
            pltpu.sync_copy(rowt, acc.at[dlt], add=True)
            plsc.subcore_barrier()

            pltpu.sync_copy(acc.at[pl.ds(s * OR, OR)],
                            out_hbms[g].at[pl.ds(lo + s * OR, OR)])
            if g != FG - 1:
                plsc.subcore_barrier()

    return k(src, dst, *y_groups)


def _row_spec(width):
    return pl.BlockSpec((RB, width), lambda i: (i, 0))


def _full_spec(shape):
    return pl.BlockSpec(shape, lambda i: tuple(0 for _ in shape))


def _tc_first(xp, w1p, dinv):
    def body(x_ref, w_ref, d_ref, y0_ref, y1_ref):
        xb = jnp.clip(x_ref[...], -0.4, 0.4)
        y = d_ref[...] * jnp.dot(xb, w_ref[...], preferred_element_type=F32)
        y0_ref[...] = y[:, :64]
        y1_ref[...] = y[:, 64:]

    return pl.pallas_call(
        body,
        grid=(NBLK,),
        in_specs=[_row_spec(128), _full_spec((128, 128)), _row_spec(1)],
        out_specs=[_row_spec(64), _row_spec(64)],
        out_shape=[jax.ShapeDtypeStruct((NP, 64), F32)] * 2,
    )(xp, w1p, dinv)


def _tc_mid(s_groups, y_groups, dinv, bp, wnextp, fg_out):
    fg_in = len(s_groups)
    dout = 64 * fg_out

    def body(*refs):
        s_refs = refs[:fg_in]
        y_refs = refs[fg_in:2 * fg_in]
        d_ref, b_ref, w_ref = refs[2 * fg_in:2 * fg_in + 3]
        out_refs = refs[2 * fg_in + 3:]
        sb = jnp.concatenate([r[...] for r in s_refs], axis=1)
        yb = jnp.concatenate([r[...] for r in y_refs], axis=1)
        h = jax.nn.relu(d_ref[...] * (sb + yb) + b_ref[...])
        z = d_ref[...] * jnp.dot(h, w_ref[...], preferred_element_type=F32)
        for g in range(fg_out):
            out_refs[g][...] = z[:, g * 64:(g + 1) * 64]

    return pl.pallas_call(
        body,
        grid=(NBLK,),
        in_specs=([_row_spec(64)] * (2 * fg_in)
                  + [_row_spec(1), _full_spec((1, 64 * fg_in)),
                     _full_spec((64 * fg_in, dout))]),
        out_specs=[_row_spec(64)] * fg_out,
        out_shape=[jax.ShapeDtypeStruct((NP, 64), F32)] * fg_out,
    )(*s_groups, *y_groups, dinv, bp, wnextp)


def _tc_head1(s_groups, y_groups, dinv, b3p, wf1, bf1, wf2, bf2):
    fg = len(s_groups)

    def body(*refs):
        s_refs = refs[:fg]
        y_refs = refs[fg:2 * fg]
        d_ref, b3_ref, w1_ref, b1_ref, w2_ref, b2_ref = refs[2 * fg:2 * fg + 6]
        f2_ref, st_ref, accum = refs[2 * fg + 6:]
        i = pl.program_id(0)

        @pl.when(i == 0)
        def _():
            accum[...] = jnp.zeros((8, 128), F32)

        sb = jnp.concatenate([r[...] for r in s_refs], axis=1)
        yb = jnp.concatenate([r[...] for r in y_refs], axis=1)
        h3 = jax.nn.relu(d_ref[...] * (sb + yb) + b3_ref[...])
        f1 = jax.nn.relu(
            jnp.dot(h3, w1_ref[...], preferred_element_type=F32) + b1_ref[...])
        f2 = jnp.dot(f1, w2_ref[...], preferred_element_type=F32) + b2_ref[...]
        f2_ref[...] = f2
        rows = i * RB + lax.broadcasted_iota(I32, (RB, 128), 0)
        f2m = jnp.where(rows < N, f2, 0.0)
        ssum = jnp.sum(f2m, axis=0, keepdims=True)
        ssq = jnp.sum(f2m * f2m, axis=0, keepdims=True)
        accum[0:1, :] = accum[0:1, :] + ssum
        accum[1:2, :] = accum[1:2, :] + ssq

        @pl.when(i == NBLK - 1)
        def _():
            st_ref[...] = accum[...]

    return pl.pallas_call(
        body,
        grid=(NBLK,),
        in_specs=([_row_spec(64)] * (2 * fg)
                  + [_row_spec(1), _full_spec((1, 256)),
                     _full_spec((256, 256)), _full_spec((1, 256)),
                     _full_spec((256, 128)), _full_spec((1, 128))]),
        out_specs=[_row_spec(128), _full_spec((8, 128))],
        out_shape=[jax.ShapeDtypeStruct((NP, 128), F32),
                   jax.ShapeDtypeStruct((8, 128), F32)],
        scratch_shapes=[pltpu.VMEM((8, 128), F32)],
    )(*s_groups, *y_groups, dinv, b3p, wf1, bf1, wf2, bf2)


def _tc_head2(f2, stats, gammap, betap, wf3p, bf3p):
    def body(f_ref, st_ref, g_ref, be_ref, w_ref, b_ref, o_ref):
        mean = st_ref[0:1, :] * (1.0 / N)
        var = st_ref[1:2, :] * (1.0 / N) - mean * mean
        xn = (f_ref[...] - mean) * lax.rsqrt(var + EPS) * g_ref[...] + be_ref[...]
        r = jax.nn.relu(xn)
        z = jnp.dot(r, w_ref[...], preferred_element_type=F32) + b_ref[...]
        col = lax.broadcasted_iota(I32, (RB, 128), 1)
        zm = jnp.where(col < 19, z, -1e30)
        mx = jnp.max(zm, axis=1, keepdims=True)
        lse = jnp.log(jnp.sum(jnp.exp(zm - mx), axis=1, keepdims=True))
        o_ref[...] = zm - mx - lse

    return pl.pallas_call(
        body,
        grid=(NBLK,),
        in_specs=[_row_spec(128), _full_spec((8, 128)), _full_spec((1, 128)),
                  _full_spec((1, 128)), _full_spec((128, 128)),
                  _full_spec((1, 128))],
        out_specs=_row_spec(128),
        out_shape=jax.ShapeDtypeStruct((NP, 128), F32),
    )(f2, stats, gammap, betap, wf3p, bf3p)


def _pad2(a, r, c):
    return jnp.pad(a, ((0, r - a.shape[0]), (0, c - a.shape[1])))


def kernel(x, edge_index, W1, b1, W2, b2, W3, b3, Wf1, bf1, Wf2, bf2,
           gamma, beta, Wf3, bf3):
    src = edge_index[0]
    dst = edge_index[1]

    xp = _pad2(x, NP, 128)
    w1p = _pad2(W1, 128, 128)
    w2p = _pad2(W2, 128, 128)
    w3p = _pad2(W3, 128, 256)
    b1p = jnp.pad(b1, (0, 28)).reshape(1, 128)
    b2p = jnp.pad(b2, (0, 28)).reshape(1, 128)
    b3p = b3.reshape(1, 256)
    bf1p = bf1.reshape(1, 256)
    bf2p = bf2.reshape(1, 128)
    gammap = gamma.reshape(1, 128)
    betap = beta.reshape(1, 128)
    wf3p = _pad2(Wf3, 128, 128)
    bf3p = jnp.pad(bf3, (0, 109)).reshape(1, 128)

    dinv = _sc_degree(dst).reshape(NP, 1)

    y1 = _tc_first(xp, w1p, dinv)
    s1 = _sc_conv(src, dst, y1)
    y2 = _tc_mid(s1, y1, dinv, b1p, w2p, 2)
    s2 = _sc_conv(src, dst, y2)
    y3 = _tc_mid(s2, y2, dinv, b2p, w3p, 4)
    s3 = _sc_conv(src, dst, y3)
    f2, stats = _tc_head1(s3, y3, dinv, b3p, Wf1, bf1p, Wf2, bf2p)
    out = _tc_head2(f2, stats, gammap, betap, wf3p, bf3p)
    return out[:N, :19]


# speedup vs baseline: 9.2379x; 1.3862x over previous
import dataclasses
import functools

import jax
import jax.numpy as jnp
from jax import lax
from jax.experimental import pallas as pl
from jax.experimental.pallas import tpu as pltpu
from jax.experimental.pallas import tpu_sc as plsc

F32 = jnp.float32
I32 = jnp.int32

N = 50000
E = 800000
NP = 50176
NPC = NP // 2
ACC_ROWS = NPC + 2
DUMP = NPC
NSC = 16
EPS = 1e-5
W = 128
RB = 512
NBLK = NP // RB


def _mesh():
    return plsc.VectorSubcoreMesh(core_axis_name="c", subcore_axis_name="s")


def _sc_params():
    return pltpu.CompilerParams(needs_layout_passes=False,
                                use_tc_tiling_on_sc=False)


def _zero_rows(ref, nrows):
    @pl.loop(0, nrows)
    def _(i):
        @pl.loop(0, 4)
        def _(j):
            ref[i, pl.ds(j * 16, 16)] = jnp.zeros((16,), F32)


def _sc_degree(dst):
    EC = E // NSC
    NWIN = EC // W
    TAIL = EC - NWIN * W
    SLC = NPC // NSC

    @functools.partial(
        pl.kernel,
        out_type=jax.ShapeDtypeStruct((NP,), F32),
        mesh=_mesh(),
        compiler_params=_sc_params(),
        scratch_types=[
            pltpu.VMEM((W,), I32),
            pltpu.VMEM((TAIL,), I32),
            pltpu.VMEM((W,), F32),
            pltpu.VMEM((SLC,), F32),
            pltpu.VMEM((SLC,), F32),
            pltpu.VMEM_SHARED((NP,), F32),
        ],
    )
    def k(dst_hbm, dinv_hbm, idxb, idxt, oneb, cntb, outb, acc):
        c = lax.axis_index("c")
        s = lax.axis_index("s")
        @pl.loop(0, W // 16)
        def _(i):
            oneb[pl.ds(i * 16, 16)] = jnp.ones((16,), F32)

        @pl.loop(0, SLC // 16)
        def _(i):
            cntb[pl.ds(i * 16, 16)] = jnp.zeros((16,), F32)

        pltpu.sync_copy(cntb, acc.at[pl.ds(s * (NP // NSC), SLC)])
        pltpu.sync_copy(cntb, acc.at[pl.ds(s * (NP // NSC) + SLC, NP // NSC - SLC)])
        plsc.subcore_barrier()

        base = s * EC

        @pl.loop(0, NWIN)
        def _(w):
            pltpu.sync_copy(dst_hbm.at[pl.ds(base + w * W, W)], idxb)
            pltpu.sync_copy(oneb, acc.at[idxb], add=True)

        pltpu.sync_copy(dst_hbm.at[pl.ds(base + NWIN * W, TAIL)], idxt)
        pltpu.sync_copy(oneb.at[pl.ds(0, TAIL)], acc.at[idxt], add=True)
        plsc.subcore_barrier()

        r0 = c * NPC + s * SLC
        pltpu.sync_copy(acc.at[pl.ds(r0, SLC)], cntb)

        @pl.loop(0, SLC // 16)
        def _(i):
            d = cntb[pl.ds(i * 16, 16)] + 1.0
            iv = plsc.bitcast(d, I32)
            iv = 0x5F3759DF - lax.shift_right_arithmetic(iv, 1)
            y = plsc.bitcast(iv, F32)
            y = y * (1.5 - 0.5 * d * y * y)
            y = y * (1.5 - 0.5 * d * y * y)
            y = y * (1.5 - 0.5 * d * y * y)
            outb[pl.ds(i * 16, 16)] = y

        pltpu.sync_copy(outb, dinv_hbm.at[pl.ds(r0, SLC)])

    return k(dst)


CAP = 25344
EBLK = 2048


def _sc_compact(src, dst):
    EC2 = E // 32
    NBLKE = EC2 // EBLK
    TAILE = EC2 - NBLKE * EBLK

    @functools.partial(
        pl.kernel,
        out_type=[jax.ShapeDtypeStruct((64, CAP), I32),
                  jax.ShapeDtypeStruct((64, CAP), I32),
                  jax.ShapeDtypeStruct((32, 16), I32)],
        mesh=_mesh(),
        compiler_params=_sc_params(),
        scratch_types=[
            pltpu.VMEM((EBLK,), I32),
            pltpu.VMEM((EBLK,), I32),
            pltpu.VMEM((CAP,), I32),
            pltpu.VMEM((CAP,), I32),
            pltpu.VMEM((CAP,), I32),
            pltpu.VMEM((CAP,), I32),
            pltpu.VMEM((16,), I32),
        ],
    )
    def k(src_hbm, dst_hbm, srcc, dlc, cnts, srcin, dstin,
          s0b, d0b, s1b, d1b, cbuf):
        c = lax.axis_index("c")
        s = lax.axis_index("s")
        t = s * 2 + c
        base = t * EC2
        iota = lax.iota(I32, 16)
        zero16 = jnp.zeros((16,), I32)

        def vec_body(i, ptrs, valid=None):
            p0, p1 = ptrs
            sv = srcin[pl.ds(i * 16, 16)]
            dv = dstin[pl.ds(i * 16, 16)]
            m0 = dv < NPC
            if valid is not None:
                m0 = m0 & valid
            mi = jnp.where(m0, 1, 0)
            cs = lax.cumsum(mi, axis=0)
            pos0 = p0 + (cs - mi)
            plsc.store_scatter(s0b, [pos0], sv, mask=m0)
            plsc.store_scatter(d0b, [pos0], dv, mask=m0)
            p0 = p0 + plsc.all_reduce_population_count(m0)
            m1 = jnp.logical_not(dv < NPC)
            if valid is not None:
                m1 = m1 & valid
            mi1 = jnp.where(m1, 1, 0)
            cs1 = lax.cumsum(mi1, axis=0)
            pos1 = p1 + (cs1 - mi1)
            plsc.store_scatter(s1b, [pos1], sv, mask=m1)
            plsc.store_scatter(d1b, [pos1], dv - NPC, mask=m1)
            p1 = p1 + plsc.all_reduce_population_count(m1)
            return p0, p1

        @pl.loop(0, NBLKE, init_carry=(zero16, zero16))
        def ptrs(blk, ptrs):
            pltpu.sync_copy(src_hbm.at[pl.ds(base + blk * EBLK, EBLK)], srcin)
            pltpu.sync_copy(dst_hbm.at[pl.ds(base + blk * EBLK, EBLK)], dstin)

            @pl.loop(0, EBLK // 16, init_carry=ptrs)
            def ptrs(i, ptrs):
                return vec_body(i, ptrs)

            return ptrs

        pltpu.sync_copy(src_hbm.at[pl.ds(base + NBLKE * EBLK, TAILE)],
                        srcin.at[pl.ds(0, TAILE)])
        pltpu.sync_copy(dst_hbm.at[pl.ds(base + NBLKE * EBLK, TAILE)],
                        dstin.at[pl.ds(0, TAILE)])

        @pl.loop(0, TAILE // 16, init_carry=ptrs)
        def ptrs(i, ptrs):
            return vec_body(i, ptrs)

        LEFT = TAILE - (TAILE // 16) * 16
        if LEFT:
            ptrs = vec_body(TAILE // 16, ptrs, valid=iota < LEFT)

        p0, p1 = ptrs
        dump16 = jnp.full((16,), DUMP, I32)
        for j in range(16):
            plsc.store_scatter(s0b, [p0 + iota + 16 * j], zero16)
            plsc.store_scatter(d0b, [p0 + iota + 16 * j], dump16)
            plsc.store_scatter(s1b, [p1 + iota + 16 * j], zero16)
            plsc.store_scatter(d1b, [p1 + iota + 16 * j], dump16)

        nw0 = ((p0 >> 8) + 1) * 2
        nw1 = ((p1 >> 8) + 1) * 2
        cv = jnp.where(iota == 0, nw0, 0) + jnp.where(iota == 1, nw1, 0)
        cbuf[pl.ds(0, 16)] = cv
        pltpu.sync_copy(cbuf, cnts.at[t])
        pltpu.sync_copy(s0b, srcc.at[t])
        pltpu.sync_copy(d0b, dlc.at[t])
        pltpu.sync_copy(s1b, srcc.at[32 + t])
        pltpu.sync_copy(d1b, dlc.at[32 + t])

    return k(src, dst)


def _sc_conv(srcc, dlc, cnts, y_groups):
    FG = len(y_groups)
    OR = NPC // NSC

    @functools.partial(
        pl.kernel,
        out_type=[jax.ShapeDtypeStruct((NP, 64), F32) for _ in range(FG)],
        mesh=_mesh(),
        compiler_params=_sc_params(),
        scratch_types=[
            pltpu.VMEM((2, W), I32),
            pltpu.VMEM((2, W), I32),
            pltpu.VMEM((2, W), I32),
            pltpu.VMEM((2, W, 64), F32),
            pltpu.VMEM((16,), I32),
            pltpu.VMEM((W, 64), F32),
            pltpu.VMEM_SHARED((ACC_ROWS, 64), F32),
            pltpu.SemaphoreType.DMA,
            pltpu.SemaphoreType.DMA,
            pltpu.SemaphoreType.DMA,
            pltpu.SemaphoreType.DMA,
            pltpu.SemaphoreType.DMA,
            pltpu.SemaphoreType.DMA,
        ],
    )
    def k(srcc_hbm, dlc_hbm, cnts_hbm, *rest):
        y_hbms = rest[:FG]
        out_hbms = rest[FG:2 * FG]
        (srcb, dlb, dlscat, rowb, cntb, zrow, acc,
         i0, i1, g0, g1, s0, s1) = rest[2 * FG:]
        isems, gsems, ssems = (i0, i1), (g0, g1), (s0, s1)
        c = lax.axis_index("c")
        s = lax.axis_index("s")
        lo = c * NPC
        iota = lax.iota(I32, 16)

        _zero_rows(zrow, W)

        def idx_start(seg, w, b):
            pltpu.async_copy(srcc_hbm.at[seg, pl.ds(w * W, W)],
                             srcb.at[b], isems[b])
            pltpu.async_copy(dlc_hbm.at[seg, pl.ds(w * W, W)],
                             dlb.at[b], isems[b])

        def idx_wait(seg, b):
            pltpu.make_async_copy(srcc_hbm.at[seg, pl.ds(0, W)],
                                  srcb.at[b], isems[b]).wait()
            pltpu.make_async_copy(dlc_hbm.at[seg, pl.ds(0, W)],
                                  dlb.at[b], isems[b]).wait()

        def scat_wait(b):
            pltpu.make_async_copy(rowb.at[b], acc.at[dlscat.at[b]],
                                  ssems[b]).wait()

        for g in range(FG):
            z0 = s * OR
            nfull = OR // W

            @pl.loop(0, nfull)
            def _(j, z0=z0):
                pltpu.sync_copy(zrow, acc.at[pl.ds(z0 + j * W, W)])

            rem = OR - nfull * W
            pltpu.sync_copy(zrow.at[pl.ds(0, rem)],
                            acc.at[pl.ds(z0 + nfull * W, rem)])
            plsc.subcore_barrier()

            for lsub in (0, 1):
                t = 2 * s + lsub
                seg = c * 32 + t
                pltpu.sync_copy(cnts_hbm.at[t], cntb)
                cv = cntb[pl.ds(0, 16)]
                nw = lax.reduce_max(jnp.where(iota == c, cv, 0), axes=(0,))
                npairs = lax.shift_right_logical(nw, 1)

                for b in (0, 1):
                    @pl.loop(0, W // 16)
                    def _(i, b=b):
                        dlscat[b, pl.ds(i * 16, 16)] = jnp.full((16,), DUMP,
                                                                I32)

                    pltpu.async_copy(zrow, acc.at[dlscat.at[b]], ssems[b],
                                     add=True)
                    idx_start(seg, b, b)

                @pl.loop(0, npairs)
                def _(p, g=g, seg=seg, nw=nw):
                    for b in (0, 1):
                        w = 2 * p + b
                        idx_wait(seg, b)
                        scat_wait(b)

                        @pl.loop(0, W // 16)
                        def _(i, b=b):
                            dlscat[b, pl.ds(i * 16, 16)] = dlb[
                                b, pl.ds(i * 16, 16)]

                        pltpu.async_copy(y_hbms[g].at[srcb.at[b]],
                                         rowb.at[b], gsems[b])
                        pltpu.make_async_copy(y_hbms[g].at[srcb.at[b]],
                                              rowb.at[b], gsems[b]).wait()

                        @pl.when(w + 2 < nw)
                        def _(w=w, b=b):
                            idx_start(seg, w + 2, b)

                        pltpu.async_copy(rowb.at[b], acc.at[dlscat.at[b]],
                                         ssems[b], add=True)

                scat_wait(0)
                scat_wait(1)

            plsc.subcore_barrier()

            pltpu.sync_copy(acc.at[pl.ds(s * OR, OR)],
                            out_hbms[g].at[pl.ds(lo + s * OR, OR)])
            if g != FG - 1:
                plsc.subcore_barrier()

    return k(srcc, dlc, cnts, *y_groups)


def _row_spec(width):
    return pl.BlockSpec((RB, width), lambda i: (i, 0))


def _full_spec(shape):
    return pl.BlockSpec(shape, lambda i: tuple(0 for _ in shape))


def _tc_first(xp, w1p, dinv):
    def body(x_ref, w_ref, d_ref, y0_ref, y1_ref):
        xb = jnp.clip(x_ref[...], -0.4, 0.4)
        y = d_ref[...] * jnp.dot(xb, w_ref[...], preferred_element_type=F32)
        y0_ref[...] = y[:, :64]
        y1_ref[...] = y[:, 64:]

    return pl.pallas_call(
        body,
        grid=(NBLK,),
        in_specs=[_row_spec(128), _full_spec((128, 128)), _row_spec(1)],
        out_specs=[_row_spec(64), _row_spec(64)],
        out_shape=[jax.ShapeDtypeStruct((NP, 64), F32)] * 2,
    )(xp, w1p, dinv)


def _tc_mid(s_groups, y_groups, dinv, bp, wnextp, fg_out):
    fg_in = len(s_groups)
    dout = 64 * fg_out

    def body(*refs):
        s_refs = refs[:fg_in]
        y_refs = refs[fg_in:2 * fg_in]
        d_ref, b_ref, w_ref = refs[2 * fg_in:2 * fg_in + 3]
        out_refs = refs[2 * fg_in + 3:]
        sb = jnp.concatenate([r[...] for r in s_refs], axis=1)
        yb = jnp.concatenate([r[...] for r in y_refs], axis=1)
        h = jax.nn.relu(d_ref[...] * (sb + yb) + b_ref[...])
        z = d_ref[...] * jnp.dot(h, w_ref[...], preferred_element_type=F32)
        for g in range(fg_out):
            out_refs[g][...] = z[:, g * 64:(g + 1) * 64]

    return pl.pallas_call(
        body,
        grid=(NBLK,),
        in_specs=([_row_spec(64)] * (2 * fg_in)
                  + [_row_spec(1), _full_spec((1, 64 * fg_in)),
                     _full_spec((64 * fg_in, dout))]),
        out_specs=[_row_spec(64)] * fg_out,
        out_shape=[jax.ShapeDtypeStruct((NP, 64), F32)] * fg_out,
    )(*s_groups, *y_groups, dinv, bp, wnextp)


def _tc_head1(s_groups, y_groups, dinv, b3p, wf1, bf1, wf2, bf2):
    fg = len(s_groups)

    def body(*refs):
        s_refs = refs[:fg]
        y_refs = refs[fg:2 * fg]
        d_ref, b3_ref, w1_ref, b1_ref, w2_ref, b2_ref = refs[2 * fg:2 * fg + 6]
        f2_ref, st_ref, accum = refs[2 * fg + 6:]
        i = pl.program_id(0)

        @pl.when(i == 0)
        def _():
            accum[...] = jnp.zeros((8, 128), F32)

        sb = jnp.concatenate([r[...] for r in s_refs], axis=1)
        yb = jnp.concatenate([r[...] for r in y_refs], axis=1)
        h3 = jax.nn.relu(d_ref[...] * (sb + yb) + b3_ref[...])
        f1 = jax.nn.relu(
            jnp.dot(h3, w1_ref[...], preferred_element_type=F32) + b1_ref[...])
        f2 = jnp.dot(f1, w2_ref[...], preferred_element_type=F32) + b2_ref[...]
        f2_ref[...] = f2
        rows = i * RB + lax.broadcasted_iota(I32, (RB, 128), 0)
        f2m = jnp.where(rows < N, f2, 0.0)
        ssum = jnp.sum(f2m, axis=0, keepdims=True)
        ssq = jnp.sum(f2m * f2m, axis=0, keepdims=True)
        accum[0:1, :] = accum[0:1, :] + ssum
        accum[1:2, :] = accum[1:2, :] + ssq

        @pl.when(i == NBLK - 1)
        def _():
            st_ref[...] = accum[...]

    return pl.pallas_call(
        body,
        grid=(NBLK,),
        in_specs=([_row_spec(64)] * (2 * fg)
                  + [_row_spec(1), _full_spec((1, 256)),
                     _full_spec((256, 256)), _full_spec((1, 256)),
                     _full_spec((256, 128)), _full_spec((1, 128))]),
        out_specs=[_row_spec(128), _full_spec((8, 128))],
        out_shape=[jax.ShapeDtypeStruct((NP, 128), F32),
                   jax.ShapeDtypeStruct((8, 128), F32)],
        scratch_shapes=[pltpu.VMEM((8, 128), F32)],
    )(*s_groups, *y_groups, dinv, b3p, wf1, bf1, wf2, bf2)


def _tc_head2(f2, stats, gammap, betap, wf3p, bf3p):
    def body(f_ref, st_ref, g_ref, be_ref, w_ref, b_ref, o_ref):
        mean = st_ref[0:1, :] * (1.0 / N)
        var = st_ref[1:2, :] * (1.0 / N) - mean * mean
        xn = (f_ref[...] - mean) * lax.rsqrt(var + EPS) * g_ref[...] + be_ref[...]
        r = jax.nn.relu(xn)
        z = jnp.dot(r, w_ref[...], preferred_element_type=F32) + b_ref[...]
        col = lax.broadcasted_iota(I32, (RB, 128), 1)
        zm = jnp.where(col < 19, z, -1e30)
        mx = jnp.max(zm, axis=1, keepdims=True)
        lse = jnp.log(jnp.sum(jnp.exp(zm - mx), axis=1, keepdims=True))
        o_ref[...] = zm - mx - lse

    return pl.pallas_call(
        body,
        grid=(NBLK,),
        in_specs=[_row_spec(128), _full_spec((8, 128)), _full_spec((1, 128)),
                  _full_spec((1, 128)), _full_spec((128, 128)),
                  _full_spec((1, 128))],
        out_specs=_row_spec(128),
        out_shape=jax.ShapeDtypeStruct((NP, 128), F32),
    )(f2, stats, gammap, betap, wf3p, bf3p)


def _pad2(a, r, c):
    return jnp.pad(a, ((0, r - a.shape[0]), (0, c - a.shape[1])))


def kernel(x, edge_index, W1, b1, W2, b2, W3, b3, Wf1, bf1, Wf2, bf2,
           gamma, beta, Wf3, bf3):
    src = edge_index[0]
    dst = edge_index[1]

    xp = _pad2(x, NP, 128)
    w1p = _pad2(W1, 128, 128)
    w2p = _pad2(W2, 128, 128)
    w3p = _pad2(W3, 128, 256)
    b1p = jnp.pad(b1, (0, 28)).reshape(1, 128)
    b2p = jnp.pad(b2, (0, 28)).reshape(1, 128)
    b3p = b3.reshape(1, 256)
    bf1p = bf1.reshape(1, 256)
    bf2p = bf2.reshape(1, 128)
    gammap = gamma.reshape(1, 128)
    betap = beta.reshape(1, 128)
    wf3p = _pad2(Wf3, 128, 128)
    bf3p = jnp.pad(bf3, (0, 109)).reshape(1, 128)

    dinv = _sc_degree(dst).reshape(NP, 1)
    srcc, dlc, cnts = _sc_compact(src, dst)

    y1 = _tc_first(xp, w1p, dinv)
    s1 = _sc_conv(srcc, dlc, cnts, y1)
    y2 = _tc_mid(s1, y1, dinv, b1p, w2p, 2)
    s2 = _sc_conv(srcc, dlc, cnts, y2)
    y3 = _tc_mid(s2, y2, dinv, b2p, w3p, 4)
    s3 = _sc_conv(srcc, dlc, cnts, y3)
    f2, stats = _tc_head1(s3, y3, dinv, b3p, Wf1, bf1p, Wf2, bf2p)
    out = _tc_head2(f2, stats, gammap, betap, wf3p, bf3p)
    return out[:N, :19]
